# baseline (speedup 1.0000x reference)
import functools

import jax
import jax.numpy as jnp
from jax import lax
from jax.experimental import pallas as pl
from jax.experimental.pallas import tpu as pltpu
from jax.experimental.pallas import tpu_sc as plsc

N = 10000
D = 128
E = 320000

NC = 2
NS = 16
NW = NC * NS

CH = 128
NCH = 80
NIB = 4
NCH_A = NCH + NIB
EPW = NCH * CH
E_PAD = NW * EPW
N_PAD = 10240
RPW = N_PAD // NS


def _sc_agg_body(feat_hbm, idx_hbm, zeros_hbm, out_hbm,
                 acc_sh, i0, i1, i2, i3, rows0, rows1,
                 is0, is1, is2, is3, gs0, gs1):
    c = lax.axis_index("c")
    s = lax.axis_index("s")
    wid = s * NC + c

    islot = [i0, i1, i2, i3]
    isem = [is0, is1, is2, is3]
    rows = [rows0, rows1]
    gsem = [gs0, gs1]

    def idx_copy(ch, b):
        return pltpu.make_async_copy(idx_hbm.at[wid, ch], islot[b], isem[b])

    def gather(slot, b):
        return pltpu.make_async_copy(
            feat_hbm.at[islot[slot].at[0]], rows[b], gsem[b])

    for j in range(NIB):
        idx_copy(j, j).start()
    pltpu.sync_copy(zeros_hbm.at[pl.ds(s * RPW, RPW)],
                    acc_sh.at[pl.ds(s * RPW, RPW)])
    idx_copy(0, 0).wait()
    gather(0, 0).start()

    plsc.subcore_barrier()

    def body(t, carry):
        for j in range(NIB):
            ch = 4 * t + j
            b = j % 2
            idx_copy(ch + 1, (j + 1) % NIB).wait()
            gather((j + 1) % NIB, (b + 1) % 2).start()
            gather(j, b).wait()
            pltpu.sync_copy(rows[b], acc_sh.at[islot[j].at[1]], add=True)
            idx_copy(ch + NIB, j).start()
        return carry

    lax.fori_loop(0, NCH // NIB, body, 0, unroll=False)

    gather(0, 0).wait()
    idx_copy(NCH + 1, 1).wait()
    idx_copy(NCH + 2, 2).wait()
    idx_copy(NCH + 3, 3).wait()

    plsc.subcore_barrier()
    pltpu.sync_copy(acc_sh.at[pl.ds(s * RPW, RPW)],
                    out_hbm.at[c, pl.ds(s * RPW, RPW)])


def _sc_aggregate(feat, idx4, zeros):
    mesh = plsc.VectorSubcoreMesh(core_axis_name="c", subcore_axis_name="s")
    k = functools.partial(
        pl.kernel,
        mesh=mesh,
        out_type=jax.ShapeDtypeStruct((NC, N_PAD, D), jnp.float32),
        scratch_types=[
            pltpu.VMEM_SHARED((N_PAD, D), jnp.float32),
            pltpu.VMEM((2, CH), jnp.int32),
            pltpu.VMEM((2, CH), jnp.int32),
            pltpu.VMEM((2, CH), jnp.int32),
            pltpu.VMEM((2, CH), jnp.int32),
            pltpu.VMEM((CH, D), jnp.float32),
            pltpu.VMEM((CH, D), jnp.float32),
            pltpu.SemaphoreType.DMA,
            pltpu.SemaphoreType.DMA,
            pltpu.SemaphoreType.DMA,
            pltpu.SemaphoreType.DMA,
            pltpu.SemaphoreType.DMA,
            pltpu.SemaphoreType.DMA,
        ],
    )(_sc_agg_body)
    return k(feat, idx4, zeros)


def _tc_combine_body(agg_ref, feat_ref, wn_ref, ws_ref, b_ref, out_ref):
    agg = agg_ref[0] + agg_ref[1]
    out_ref[...] = (
        jnp.dot(agg, wn_ref[...], preferred_element_type=jnp.float32)
        + jnp.dot(feat_ref[...], ws_ref[...], preferred_element_type=jnp.float32)
        + b_ref[...]
    )


def _tc_combine(agg2, feat, W_neigh, b_neigh, W_self):
    BR = 1000
    grid = N // BR
    return pl.pallas_call(
        _tc_combine_body,
        grid=(grid,),
        in_specs=[
            pl.BlockSpec((NC, BR, D), lambda i: (0, i, 0)),
            pl.BlockSpec((BR, D), lambda i: (i, 0)),
            pl.BlockSpec((D, D), lambda i: (0, 0)),
            pl.BlockSpec((D, D), lambda i: (0, 0)),
            pl.BlockSpec((1, D), lambda i: (0, 0)),
        ],
        out_specs=pl.BlockSpec((BR, D), lambda i: (i, 0)),
        out_shape=jax.ShapeDtypeStruct((N, D), jnp.float32),
    )(agg2, feat, W_neigh, W_self, b_neigh.reshape(1, D))


@jax.jit
def kernel(feat, edge_index, W_neigh, b_neigh, W_self):
    src = edge_index[0].astype(jnp.int32)
    dst = edge_index[1].astype(jnp.int32)
    src3 = jnp.full((NW, NCH_A, CH), 0, jnp.int32)
    dst3 = jnp.full((NW, NCH_A, CH), N, jnp.int32)
    src3 = src3.at[:, :NCH].set(
        jnp.concatenate([src, jnp.zeros((E_PAD - E,), jnp.int32)])
        .reshape(NW, NCH, CH))
    dst3 = dst3.at[:, :NCH].set(
        jnp.concatenate([dst, jnp.full((E_PAD - E,), N, jnp.int32)])
        .reshape(NW, NCH, CH))
    idx4 = jnp.stack([src3, dst3], axis=2)
    zeros = jnp.zeros((N_PAD, D), jnp.float32)
    agg2 = _sc_aggregate(feat, idx4, zeros)
    return _tc_combine(agg2, feat, W_neigh, b_neigh, W_self)
